# Initial kernel scaffold; baseline (speedup 1.0000x reference)
import jax
import jax.numpy as jnp
from jax.experimental import pallas as pl


def kernel(x, table):
    raise NotImplementedError("write your pallas kernel here")


# speedup vs baseline: 1.3014x; 1.3014x over previous
import functools
import math

import jax
import jax.numpy as jnp
from jax import lax
from jax.experimental import pallas as pl
from jax.experimental.pallas import tpu as pltpu
from jax.experimental.pallas import tpu_sc as plsc

D_MODEL = 1024
SCALE = math.sqrt(D_MODEL)

NC = 2
NS = 16
LANES = 16
NW = NC * NS

CHUNK = 32
VECS_PER_ROW = D_MODEL // LANES


@functools.cache
def _build(B):
  n_per_w = B // NW
  n_chunks = n_per_w // CHUNK

  mesh = plsc.VectorSubcoreMesh(core_axis_name="c", subcore_axis_name="s")

  @functools.partial(
      pl.kernel,
      out_type=jax.ShapeDtypeStruct((B, D_MODEL), jnp.float32),
      mesh=mesh,
      scratch_types=[
          pltpu.VMEM((n_chunks, CHUNK), jnp.int32),
          pltpu.VMEM((CHUNK, D_MODEL), jnp.float32),
          pltpu.VMEM((CHUNK, D_MODEL), jnp.float32),
          pltpu.SemaphoreType.DMA,
          pltpu.SemaphoreType.DMA,
      ],
  )
  def emb_kernel(idx_hbm, table_hbm, out_hbm, idx_v, buf0, buf1, sem0, sem1):
    wid = lax.axis_index("s") * NC + lax.axis_index("c")
    base = wid * n_per_w

    pltpu.sync_copy(idx_hbm.at[wid], idx_v)

    bufs = (buf0, buf1)
    sems = (sem0, sem1)

    handles = [pltpu.async_copy(table_hbm.at[idx_v.at[0]], buf0, sem0)]

    for c in range(n_chunks):
      cur = bufs[c % 2]
      if c + 1 < n_chunks:
        handles.append(
            pltpu.async_copy(
                table_hbm.at[idx_v.at[c + 1]], bufs[(c + 1) % 2],
                sems[(c + 1) % 2]))
      handles[c].wait()

      def scale_row(r, _, cur=cur):
        for j in range(VECS_PER_ROW):
          cur[r, pl.ds(j * LANES, LANES)] = (
              cur[r, pl.ds(j * LANES, LANES)] * SCALE)
        return _

      lax.fori_loop(0, CHUNK, scale_row, 0, unroll=False)

      pltpu.sync_copy(cur, out_hbm.at[pl.ds(base + c * CHUNK, CHUNK)])

  return emb_kernel


def kernel(x, table):
  orig_shape = x.shape
  B = x.size
  idx = x.reshape(NW, B // NW // CHUNK, CHUNK).astype(jnp.int32)
  out = _build(B)(idx, table)
  return out.reshape(*orig_shape, D_MODEL)


# speedup vs baseline: 1.4727x; 1.1316x over previous
import functools
import math

import jax
import jax.numpy as jnp
from jax import lax
from jax.experimental import pallas as pl
from jax.experimental.pallas import tpu as pltpu
from jax.experimental.pallas import tpu_sc as plsc

D_MODEL = 1024
SCALE = math.sqrt(D_MODEL)

NC = 2
NS = 16
LANES = 16
NW = NC * NS

CHUNK = 32
VECS_PER_ROW = D_MODEL // LANES


@functools.cache
def _build(B):
  n_per_w = B // NW
  n_chunks = n_per_w // CHUNK

  mesh = plsc.VectorSubcoreMesh(core_axis_name="c", subcore_axis_name="s")

  NBUF = 3

  @functools.partial(
      pl.kernel,
      out_type=jax.ShapeDtypeStruct((B, D_MODEL), jnp.float32),
      mesh=mesh,
      scratch_types=[
          pltpu.VMEM((n_chunks, CHUNK), jnp.int32),
      ] + [pltpu.VMEM((CHUNK, D_MODEL), jnp.float32)] * NBUF
        + [pltpu.SemaphoreType.DMA] * (2 * NBUF),
  )
  def emb_kernel(idx_hbm, table_hbm, out_hbm, idx_v, *bufs_sems):
    bufs = bufs_sems[:NBUF]
    gsems = bufs_sems[NBUF:2 * NBUF]
    osems = bufs_sems[2 * NBUF:]

    wid = lax.axis_index("s") * NC + lax.axis_index("c")
    base = wid * n_per_w

    pltpu.sync_copy(idx_hbm.at[wid], idx_v)

    gather_h = [None] * n_chunks
    scatter_h = [None] * n_chunks

    def start_gather(c):
      gather_h[c] = pltpu.async_copy(
          table_hbm.at[idx_v.at[c]], bufs[c % NBUF], gsems[c % NBUF])

    for c in range(min(NBUF - 1, n_chunks)):
      start_gather(c)

    for c in range(n_chunks):
      cur = bufs[c % NBUF]
      gather_h[c].wait()

      def scale_row(r, _, cur=cur):
        for j in range(VECS_PER_ROW):
          cur[r, pl.ds(j * LANES, LANES)] = (
              cur[r, pl.ds(j * LANES, LANES)] * SCALE)
        return _

      lax.fori_loop(0, CHUNK, scale_row, 0, unroll=False)

      scatter_h[c] = pltpu.async_copy(
          cur, out_hbm.at[pl.ds(base + c * CHUNK, CHUNK)], osems[c % NBUF])

      nxt = c + NBUF - 1
      if nxt < n_chunks and gather_h[nxt] is None:
        if c >= 1:
          scatter_h[c - 1].wait()
        start_gather(nxt)

    for c in range(max(0, n_chunks - NBUF), n_chunks):
      scatter_h[c].wait()

  return emb_kernel


def kernel(x, table):
  orig_shape = x.shape
  B = x.size
  idx = x.reshape(NW, B // NW // CHUNK, CHUNK).astype(jnp.int32)
  out = _build(B)(idx, table)
  return out.reshape(*orig_shape, D_MODEL)
